# baseline (speedup 1.0000x reference)
import functools
import jax
import jax.numpy as jnp
from jax import lax
from jax.experimental import pallas as pl
from jax.experimental.pallas import tpu as pltpu
from jax.experimental.pallas import tpu_sc as plsc

D = 256
K = 10
IOU_THR = 0.5
BQ_A = 512
BQ_B = 256


def _idtoken_body(tgt_ref, w1_ref, b1_ref, w2_ref, b2_ref, g_ref, bln_ref,
                  w3_ref, id_ref, p_ref):
    x = tgt_ref[...]
    h = jnp.maximum(
        jnp.dot(x, w1_ref[...], preferred_element_type=jnp.float32) + b1_ref[...],
        0.0)
    y = jnp.dot(h, w2_ref[...], preferred_element_type=jnp.float32) + b2_ref[...]
    mu = jnp.mean(y, axis=-1, keepdims=True)
    var = jnp.mean((y - mu) ** 2, axis=-1, keepdims=True)
    idt = (y - mu) / jnp.sqrt(var + 1e-5) * g_ref[...] + bln_ref[...]
    id_ref[...] = idt
    p_ref[...] = jnp.dot(idt, w3_ref[...], preferred_element_type=jnp.float32)


def _topk_body(nq_real, bq_ref, bc_ref, sm_ref, hsm_ref, idx_ref, vals_ref):
    q = bq_ref[0]
    qx1 = q[:, 0:1] - 0.5 * q[:, 2:3]
    qy1 = q[:, 1:2] - 0.5 * q[:, 3:4]
    qx2 = q[:, 0:1] + 0.5 * q[:, 2:3]
    qy2 = q[:, 1:2] + 0.5 * q[:, 3:4]
    qarea = (qx2 - qx1) * (qy2 - qy1)

    c = bc_ref[0]
    cx1 = c[0:1, :] - 0.5 * c[2:3, :]
    cy1 = c[1:2, :] - 0.5 * c[3:4, :]
    cx2 = c[0:1, :] + 0.5 * c[2:3, :]
    cy2 = c[1:2, :] + 0.5 * c[3:4, :]
    carea = (cx2 - cx1) * (cy2 - cy1)

    iw = jnp.maximum(jnp.minimum(qx2, cx2) - jnp.maximum(qx1, cx1), 0.0)
    ih = jnp.maximum(jnp.minimum(qy2, cy2) - jnp.maximum(qy1, cy1), 0.0)
    inter = iw * ih
    iou = inter / (qarea + carea - inter + 1e-7)

    neg = 1.0 - sm_ref[0]
    ov = jnp.where(hsm_ref[0] != 0, iou * neg, 0.0)

    col = jax.lax.broadcasted_iota(jnp.int32, ov.shape, 1)
    ov = jnp.where(col < nq_real, ov, -1.0)

    nqp = ov.shape[1]
    nslab = nqp // 128
    colg = [col[:, g * 128:(g + 1) * 128] for g in range(nslab)]
    gbase = pl.program_id(0) * nqp

    def refold(x):
        v = x[:, 0:128]
        cc = colg[0]
        for g in range(1, nslab):
            vg = x[:, g * 128:(g + 1) * 128]
            take = vg > v
            v = jnp.maximum(v, vg)
            cc = jnp.where(take, colg[g], cc)
        return v, cc

    foldv, foldc = refold(ov)
    for k in range(K):
        m = jnp.max(foldv, axis=1, keepdims=True)
        am = jnp.min(jnp.where(foldv == m, foldc, nqp * 2),
                     axis=1, keepdims=True)
        sel = col == am
        ov = jnp.where(sel, -1.0, ov)
        foldv, foldc = refold(ov)
        idx_ref[0, :, k:k + 1] = am + gbase
        vals_ref[0, :, k:k + 1] = m


def _mlp_body(pj_ref, vals_ref, sm_ref, idt_ref, pq_ref, w4_ref, b3_ref,
              b4_ref, out_ref):
    pq = pq_ref[0]
    pj = pj_ref[0]
    vals = vals_ref[0]
    b3 = b3_ref[...]
    b4 = b4_ref[...]
    w4 = w4_ref[...]
    acc = jnp.full(pq.shape, -jnp.inf, jnp.float32)
    for k in range(K):
        pjk = pj[:, k * D:(k + 1) * D]
        m = vals[:, k:k + 1]
        f = jnp.maximum(pq - pjk + b3, 0.0)
        f = jnp.dot(f, w4, preferred_element_type=jnp.float32) + b4
        acc = jnp.maximum(acc, jnp.where(m >= IOU_THR, f, 0.0))
    neg = 1.0 - sm_ref[0]
    out_ref[0] = idt_ref[0] * neg + acc


def _make_sc_gather(btot, d):
    info = plsc.get_sparse_core_info()
    nc, ns = info.num_cores, info.num_subcores
    nw = nc * ns
    bpw = btot // nw
    chunk = 128
    nch = bpw // chunk
    mesh = plsc.VectorSubcoreMesh(core_axis_name="c", subcore_axis_name="s")

    @functools.partial(
        pl.kernel, mesh=mesh,
        out_type=jax.ShapeDtypeStruct((btot, d), jnp.float32),
        scratch_types=[
            pltpu.VMEM((chunk,), jnp.int32),
            pltpu.VMEM((chunk, d), jnp.float32),
            pltpu.SemaphoreType.DMA,
        ],
    )
    def gather_k(table_hbm, idx_hbm, out_hbm, idx_v, rows_v, sem):
        wid = lax.axis_index("s") * nc + lax.axis_index("c")
        base = wid * bpw
        for ch in range(nch):
            off = base + ch * chunk
            pltpu.sync_copy(idx_hbm.at[pl.ds(off, chunk)], idx_v)
            pltpu.async_copy(table_hbm.at[idx_v], rows_v, sem).wait()
            pltpu.sync_copy(rows_v, out_hbm.at[pl.ds(off, chunk)])

    return gather_k


def kernel(tgt, seed_mask, pred_boxes, attention_position, attention_weight,
           high_score_mask, prev_scores, W1, b1, W2, b2, ln_g, ln_b, W3, b3,
           W4, b4):
    bs, nq, d = tgt.shape
    assert d == D

    w1t = W1.T
    w2t = W2.T
    w3t = W3.T
    w4t = W4.T
    b1r = b1.reshape(1, d)
    b2r = b2.reshape(1, d)
    b3r = b3.reshape(1, d)
    b4r = b4.reshape(1, d)
    gr = ln_g.reshape(1, d)
    bnr = ln_b.reshape(1, d)

    nqp = 1024
    padq = nqp - nq

    tgt_flat = jnp.pad(tgt, ((0, 0), (0, padq), (0, 0))).reshape(bs * nqp, d)
    n_a = (bs * nqp) // BQ_A
    full = pl.BlockSpec((d, d), lambda i: (0, 0))
    vec = pl.BlockSpec((1, d), lambda i: (0, 0))
    id_flat, p_flat = pl.pallas_call(
        _idtoken_body,
        grid=(n_a,),
        in_specs=[
            pl.BlockSpec((BQ_A, d), lambda i: (i, 0)),
            full, vec, full, vec, vec, vec, full,
        ],
        out_specs=[
            pl.BlockSpec((BQ_A, d), lambda i: (i, 0)),
            pl.BlockSpec((BQ_A, d), lambda i: (i, 0)),
        ],
        out_shape=[
            jax.ShapeDtypeStruct((bs * nqp, d), jnp.float32),
            jax.ShapeDtypeStruct((bs * nqp, d), jnp.float32),
        ],
    )(tgt_flat, w1t, b1r, w2t, b2r, gr, bnr, w3t)

    id_p = id_flat.reshape(bs, nqp, d)
    p_p = p_flat.reshape(bs, nqp, d)

    sm_p = jnp.pad(seed_mask, ((0, 0), (0, padq), (0, 0)))
    boxes_p = jnp.pad(pred_boxes, ((0, 0), (0, padq), (0, 0)))
    boxes_c = jnp.transpose(boxes_p, (0, 2, 1))
    hsm = jnp.pad(high_score_mask[..., 0].astype(jnp.int8),
                  ((0, 0), (0, padq), (0, padq)))

    n_b = nqp // BQ_B
    idx, vals = pl.pallas_call(
        functools.partial(_topk_body, nq),
        grid=(bs, n_b),
        in_specs=[
            pl.BlockSpec((1, BQ_B, 4), lambda b, i: (b, i, 0)),
            pl.BlockSpec((1, 4, nqp), lambda b, i: (b, 0, 0)),
            pl.BlockSpec((1, BQ_B, 1), lambda b, i: (b, i, 0)),
            pl.BlockSpec((1, BQ_B, nqp), lambda b, i: (b, i, 0)),
        ],
        out_specs=[
            pl.BlockSpec((1, BQ_B, K), lambda b, i: (b, i, 0)),
            pl.BlockSpec((1, BQ_B, K), lambda b, i: (b, i, 0)),
        ],
        out_shape=[
            jax.ShapeDtypeStruct((bs, nqp, K), jnp.int32),
            jax.ShapeDtypeStruct((bs, nqp, K), jnp.float32),
        ],
    )(boxes_p, boxes_c, sm_p, hsm)

    gather_k = _make_sc_gather(bs * nqp * K, d)
    pj_flat = gather_k(p_flat, idx.reshape(bs * nqp * K))
    pj = pj_flat.reshape(bs, nqp, K * d)

    full2 = pl.BlockSpec((d, d), lambda b, i: (0, 0))
    vec2 = pl.BlockSpec((1, d), lambda b, i: (0, 0))
    out = pl.pallas_call(
        _mlp_body,
        grid=(bs, n_b),
        in_specs=[
            pl.BlockSpec((1, BQ_B, K * d), lambda b, i: (b, i, 0)),
            pl.BlockSpec((1, BQ_B, K), lambda b, i: (b, i, 0)),
            pl.BlockSpec((1, BQ_B, 1), lambda b, i: (b, i, 0)),
            pl.BlockSpec((1, BQ_B, d), lambda b, i: (b, i, 0)),
            pl.BlockSpec((1, BQ_B, d), lambda b, i: (b, i, 0)),
            full2, vec2, vec2,
        ],
        out_specs=pl.BlockSpec((1, BQ_B, d), lambda b, i: (b, i, 0)),
        out_shape=jax.ShapeDtypeStruct((bs, nqp, d), jnp.float32),
    )(pj, vals, sm_p, id_p, p_p, w4t, b3r, b4r)

    return out[:, :nq, :]


# speedup vs baseline: 2.0004x; 2.0004x over previous
import functools
import jax
import jax.numpy as jnp
from jax.experimental import pallas as pl

D = 256
K = 10
IOU_THR = 0.5
BQ_A = 512
BQ_B = 256


def _idtoken_body(tgt_ref, w1_ref, b1_ref, w2_ref, b2_ref, g_ref, bln_ref,
                  w3_ref, id_ref, p_ref):
    x = tgt_ref[...]
    h = jnp.maximum(
        jnp.dot(x, w1_ref[...], preferred_element_type=jnp.float32) + b1_ref[...],
        0.0)
    y = jnp.dot(h, w2_ref[...], preferred_element_type=jnp.float32) + b2_ref[...]
    mu = jnp.mean(y, axis=-1, keepdims=True)
    var = jnp.mean((y - mu) ** 2, axis=-1, keepdims=True)
    idt = (y - mu) / jnp.sqrt(var + 1e-5) * g_ref[...] + bln_ref[...]
    id_ref[...] = idt
    p_ref[...] = jnp.dot(idt, w3_ref[...], preferred_element_type=jnp.float32)


def _main_body(nq_real, bq_ref, bc_ref, sm_ref, hsm_ref, idt_ref, pq_ref, pf_ref,
               w4_ref, b3_ref, b4_ref, out_ref):
    q = bq_ref[0]
    qx1 = q[:, 0:1] - 0.5 * q[:, 2:3]
    qy1 = q[:, 1:2] - 0.5 * q[:, 3:4]
    qx2 = q[:, 0:1] + 0.5 * q[:, 2:3]
    qy2 = q[:, 1:2] + 0.5 * q[:, 3:4]
    qarea = (qx2 - qx1) * (qy2 - qy1)

    c = bc_ref[0]
    cx1 = c[0:1, :] - 0.5 * c[2:3, :]
    cy1 = c[1:2, :] - 0.5 * c[3:4, :]
    cx2 = c[0:1, :] + 0.5 * c[2:3, :]
    cy2 = c[1:2, :] + 0.5 * c[3:4, :]
    carea = (cx2 - cx1) * (cy2 - cy1)

    iw = jnp.maximum(jnp.minimum(qx2, cx2) - jnp.maximum(qx1, cx1), 0.0)
    ih = jnp.maximum(jnp.minimum(qy2, cy2) - jnp.maximum(qy1, cy1), 0.0)
    inter = iw * ih
    iou = inter / (qarea + carea - inter + 1e-7)

    neg = 1.0 - sm_ref[0]
    ov = jnp.where(hsm_ref[0] != 0, iou * neg, 0.0)

    col = jax.lax.broadcasted_iota(jnp.int32, ov.shape, 1)
    ov = jnp.where(col < nq_real, ov, -1.0)
    pf = pf_ref[0]
    pq = pq_ref[0]
    b3 = b3_ref[...]
    b4 = b4_ref[...]
    w4 = w4_ref[...]

    nqp = ov.shape[1]
    nslab = nqp // 128
    colg = [col[:, g * 128:(g + 1) * 128] for g in range(nslab)]

    def refold(x):
        v = x[:, 0:128]
        c = colg[0]
        for g in range(1, nslab):
            vg = x[:, g * 128:(g + 1) * 128]
            take = vg > v
            v = jnp.maximum(v, vg)
            c = jnp.where(take, colg[g], c)
        return v, c

    foldv, foldc = refold(ov)
    acc = jnp.full(pq.shape, -jnp.inf, jnp.float32)
    for _ in range(K):
        m = jnp.max(foldv, axis=1, keepdims=True)
        am = jnp.min(jnp.where(foldv == m, foldc, nqp * 2),
                     axis=1, keepdims=True)
        sel = col == am
        ov = jnp.where(sel, -1.0, ov)
        foldv, foldc = refold(ov)
        onehot = sel.astype(jnp.float32)
        pj = jnp.dot(onehot, pf, preferred_element_type=jnp.float32)
        f = jnp.maximum(pq - pj + b3, 0.0)
        f = jnp.dot(f, w4, preferred_element_type=jnp.float32) + b4
        acc = jnp.maximum(acc, jnp.where(m >= IOU_THR, f, 0.0))

    out_ref[0] = idt_ref[0] * neg + acc


def kernel(tgt, seed_mask, pred_boxes, attention_position, attention_weight,
           high_score_mask, prev_scores, W1, b1, W2, b2, ln_g, ln_b, W3, b3,
           W4, b4):
    bs, nq, d = tgt.shape
    assert d == D

    w1t = W1.T
    w2t = W2.T
    w3t = W3.T
    w4t = W4.T
    b1r = b1.reshape(1, d)
    b2r = b2.reshape(1, d)
    b3r = b3.reshape(1, d)
    b4r = b4.reshape(1, d)
    gr = ln_g.reshape(1, d)
    bnr = ln_b.reshape(1, d)

    nqp = 1024
    padq = nqp - nq

    tgt_flat = jnp.pad(tgt, ((0, 0), (0, padq), (0, 0))).reshape(bs * nqp, d)
    n_a = (bs * nqp) // BQ_A
    full = pl.BlockSpec((d, d), lambda i: (0, 0))
    vec = pl.BlockSpec((1, d), lambda i: (0, 0))
    id_flat, p_flat = pl.pallas_call(
        _idtoken_body,
        grid=(n_a,),
        in_specs=[
            pl.BlockSpec((BQ_A, d), lambda i: (i, 0)),
            full, vec, full, vec, vec, vec, full,
        ],
        out_specs=[
            pl.BlockSpec((BQ_A, d), lambda i: (i, 0)),
            pl.BlockSpec((BQ_A, d), lambda i: (i, 0)),
        ],
        out_shape=[
            jax.ShapeDtypeStruct((bs * nqp, d), jnp.float32),
            jax.ShapeDtypeStruct((bs * nqp, d), jnp.float32),
        ],
    )(tgt_flat, w1t, b1r, w2t, b2r, gr, bnr, w3t)

    id_p = id_flat.reshape(bs, nqp, d)
    p_p = p_flat.reshape(bs, nqp, d)

    sm_p = jnp.pad(seed_mask, ((0, 0), (0, padq), (0, 0)))
    boxes_p = jnp.pad(pred_boxes, ((0, 0), (0, padq), (0, 0)))
    boxes_c = jnp.transpose(boxes_p, (0, 2, 1))
    hsm = jnp.pad(high_score_mask[..., 0].astype(jnp.int8),
                  ((0, 0), (0, padq), (0, padq)))

    n_b = nqp // BQ_B
    full2 = pl.BlockSpec((d, d), lambda b, i: (0, 0))
    vec2 = pl.BlockSpec((1, d), lambda b, i: (0, 0))
    out = pl.pallas_call(
        functools.partial(_main_body, nq),
        grid=(bs, n_b),
        in_specs=[
            pl.BlockSpec((1, BQ_B, 4), lambda b, i: (b, i, 0)),
            pl.BlockSpec((1, 4, nqp), lambda b, i: (b, 0, 0)),
            pl.BlockSpec((1, BQ_B, 1), lambda b, i: (b, i, 0)),
            pl.BlockSpec((1, BQ_B, nqp), lambda b, i: (b, i, 0)),
            pl.BlockSpec((1, BQ_B, d), lambda b, i: (b, i, 0)),
            pl.BlockSpec((1, BQ_B, d), lambda b, i: (b, i, 0)),
            pl.BlockSpec((1, nqp, d), lambda b, i: (b, 0, 0)),
            full2, vec2, vec2,
        ],
        out_specs=pl.BlockSpec((1, BQ_B, d), lambda b, i: (b, i, 0)),
        out_shape=jax.ShapeDtypeStruct((bs, nqp, d), jnp.float32),
    )(boxes_p, boxes_c, sm_p, hsm, id_p, p_p, p_p, w4t, b3r, b4r)

    return out[:, :nq, :]


# speedup vs baseline: 2.0563x; 1.0280x over previous
import functools
import jax
import jax.numpy as jnp
from jax.experimental import pallas as pl

D = 256
K = 10
IOU_THR = 0.5
BQ_A = 512
BQ_B = 512


def _idtoken_body(tgt_ref, w1_ref, b1_ref, w2_ref, b2_ref, g_ref, bln_ref,
                  w3_ref, id_ref, p_ref):
    x = tgt_ref[...]
    h = jnp.maximum(
        jnp.dot(x, w1_ref[...], preferred_element_type=jnp.float32) + b1_ref[...],
        0.0)
    y = jnp.dot(h, w2_ref[...], preferred_element_type=jnp.float32) + b2_ref[...]
    mu = jnp.mean(y, axis=-1, keepdims=True)
    var = jnp.mean((y - mu) ** 2, axis=-1, keepdims=True)
    idt = (y - mu) / jnp.sqrt(var + 1e-5) * g_ref[...] + bln_ref[...]
    id_ref[...] = idt
    p_ref[...] = jnp.dot(idt, w3_ref[...], preferred_element_type=jnp.float32)


def _main_body(nq_real, bq_ref, bc_ref, sm_ref, hsm_ref, idt_ref, pq_ref, pf_ref,
               w4_ref, b3_ref, b4_ref, out_ref):
    q = bq_ref[0]
    qx1 = q[:, 0:1] - 0.5 * q[:, 2:3]
    qy1 = q[:, 1:2] - 0.5 * q[:, 3:4]
    qx2 = q[:, 0:1] + 0.5 * q[:, 2:3]
    qy2 = q[:, 1:2] + 0.5 * q[:, 3:4]
    qarea = (qx2 - qx1) * (qy2 - qy1)

    c = bc_ref[0]
    cx1 = c[0:1, :] - 0.5 * c[2:3, :]
    cy1 = c[1:2, :] - 0.5 * c[3:4, :]
    cx2 = c[0:1, :] + 0.5 * c[2:3, :]
    cy2 = c[1:2, :] + 0.5 * c[3:4, :]
    carea = (cx2 - cx1) * (cy2 - cy1)

    iw = jnp.maximum(jnp.minimum(qx2, cx2) - jnp.maximum(qx1, cx1), 0.0)
    ih = jnp.maximum(jnp.minimum(qy2, cy2) - jnp.maximum(qy1, cy1), 0.0)
    inter = iw * ih
    iou = inter / (qarea + carea - inter + 1e-7)

    neg = 1.0 - sm_ref[0]
    ov = jnp.where(hsm_ref[0] != 0, iou * neg, 0.0)

    col = jax.lax.broadcasted_iota(jnp.int32, ov.shape, 1)
    ov = jnp.where(col < nq_real, ov, -1.0)
    pf = pf_ref[0]
    pq = pq_ref[0]
    b3 = b3_ref[...]
    b4 = b4_ref[...]
    w4 = w4_ref[...]

    nqp = ov.shape[1]
    nslab = nqp // 128
    colg = [col[:, g * 128:(g + 1) * 128] for g in range(nslab)]

    def refold(x):
        v = x[:, 0:128]
        c = colg[0]
        for g in range(1, nslab):
            vg = x[:, g * 128:(g + 1) * 128]
            take = vg > v
            v = jnp.maximum(v, vg)
            c = jnp.where(take, colg[g], c)
        return v, c

    foldv, foldc = refold(ov)
    acc = jnp.full(pq.shape, -jnp.inf, jnp.float32)
    for _ in range(K):
        m = jnp.max(foldv, axis=1, keepdims=True)
        am = jnp.min(jnp.where(foldv == m, foldc, nqp * 2),
                     axis=1, keepdims=True)
        sel = col == am
        ov = jnp.where(sel, -1.0, ov)
        foldv, foldc = refold(ov)
        onehot = sel.astype(jnp.float32)
        pj = jnp.dot(onehot, pf, preferred_element_type=jnp.float32)
        f = jnp.maximum(pq - pj + b3, 0.0)
        f = jnp.dot(f, w4, preferred_element_type=jnp.float32) + b4
        acc = jnp.maximum(acc, jnp.where(m >= IOU_THR, f, 0.0))

    out_ref[0] = idt_ref[0] * neg + acc


def kernel(tgt, seed_mask, pred_boxes, attention_position, attention_weight,
           high_score_mask, prev_scores, W1, b1, W2, b2, ln_g, ln_b, W3, b3,
           W4, b4):
    bs, nq, d = tgt.shape
    assert d == D

    w1t = W1.T
    w2t = W2.T
    w3t = W3.T
    w4t = W4.T
    b1r = b1.reshape(1, d)
    b2r = b2.reshape(1, d)
    b3r = b3.reshape(1, d)
    b4r = b4.reshape(1, d)
    gr = ln_g.reshape(1, d)
    bnr = ln_b.reshape(1, d)

    nqp = 1024
    padq = nqp - nq

    tgt_flat = jnp.pad(tgt, ((0, 0), (0, padq), (0, 0))).reshape(bs * nqp, d)
    n_a = (bs * nqp) // BQ_A
    full = pl.BlockSpec((d, d), lambda i: (0, 0))
    vec = pl.BlockSpec((1, d), lambda i: (0, 0))
    id_flat, p_flat = pl.pallas_call(
        _idtoken_body,
        grid=(n_a,),
        in_specs=[
            pl.BlockSpec((BQ_A, d), lambda i: (i, 0)),
            full, vec, full, vec, vec, vec, full,
        ],
        out_specs=[
            pl.BlockSpec((BQ_A, d), lambda i: (i, 0)),
            pl.BlockSpec((BQ_A, d), lambda i: (i, 0)),
        ],
        out_shape=[
            jax.ShapeDtypeStruct((bs * nqp, d), jnp.float32),
            jax.ShapeDtypeStruct((bs * nqp, d), jnp.float32),
        ],
    )(tgt_flat, w1t, b1r, w2t, b2r, gr, bnr, w3t)

    id_p = id_flat.reshape(bs, nqp, d)
    p_p = p_flat.reshape(bs, nqp, d)

    sm_p = jnp.pad(seed_mask, ((0, 0), (0, padq), (0, 0)))
    boxes_p = jnp.pad(pred_boxes, ((0, 0), (0, padq), (0, 0)))
    boxes_c = jnp.transpose(boxes_p, (0, 2, 1))
    hsm = jnp.pad(high_score_mask[..., 0].astype(jnp.int8),
                  ((0, 0), (0, padq), (0, padq)))

    n_b = nqp // BQ_B
    full2 = pl.BlockSpec((d, d), lambda b, i: (0, 0))
    vec2 = pl.BlockSpec((1, d), lambda b, i: (0, 0))
    out = pl.pallas_call(
        functools.partial(_main_body, nq),
        grid=(bs, n_b),
        in_specs=[
            pl.BlockSpec((1, BQ_B, 4), lambda b, i: (b, i, 0)),
            pl.BlockSpec((1, 4, nqp), lambda b, i: (b, 0, 0)),
            pl.BlockSpec((1, BQ_B, 1), lambda b, i: (b, i, 0)),
            pl.BlockSpec((1, BQ_B, nqp), lambda b, i: (b, i, 0)),
            pl.BlockSpec((1, BQ_B, d), lambda b, i: (b, i, 0)),
            pl.BlockSpec((1, BQ_B, d), lambda b, i: (b, i, 0)),
            pl.BlockSpec((1, nqp, d), lambda b, i: (b, 0, 0)),
            full2, vec2, vec2,
        ],
        out_specs=pl.BlockSpec((1, BQ_B, d), lambda b, i: (b, i, 0)),
        out_shape=jax.ShapeDtypeStruct((bs, nqp, d), jnp.float32),
    )(boxes_p, boxes_c, sm_p, hsm, id_p, p_p, p_p, w4t, b3r, b4r)

    return out[:, :nq, :]
